# Initial kernel scaffold; baseline (speedup 1.0000x reference)
import functools

import jax
import jax.numpy as jnp
from jax import lax
from jax.experimental import pallas as pl
from jax.experimental.pallas import tpu as pltpu
from jax.experimental.pallas import tpu_sc as plsc

N = 10000
E = 320000
D = 128
H = 128
LAT = 64
G = 20
NL = 12
NENC = 9
NREC = 11

K = 80
TILES = 16
EPT = E // TILES
NITER = EPT // K
ROWS_PT = N // TILES

_sc_mesh = plsc.VectorSubcoreMesh(core_axis_name="c", subcore_axis_name="s")


def _sc_segsum_body(x_flat, edges, zrow, zcnt, ones_k,
                    agg_out, cnt_out,
                    agg_sh, cnt_sh, src_v, dst_v, rows_v, ones_v, gsem):
    c = lax.axis_index("c")
    s = lax.axis_index("s")
    ebase = s * EPT
    rbase = s * ROWS_PT

    def run_layer(l):
        pltpu.sync_copy(zrow.at[pl.ds(rbase, ROWS_PT)],
                        agg_sh.at[pl.ds(rbase, ROWS_PT)])
        plsc.subcore_barrier()

        def chunk(it, carry):
            base = ebase + it * K
            pltpu.sync_copy(edges.at[0, pl.ds(base, K)], src_v)
            pltpu.sync_copy(edges.at[1, pl.ds(base, K)], dst_v)
            for j in range(K // 16):
                sl = pl.ds(j * 16, 16)
                src_v[sl] = src_v[sl] + l * N
            pltpu.async_copy(x_flat.at[src_v], rows_v, gsem).wait()
            pltpu.sync_copy(rows_v, agg_sh.at[dst_v], add=True)
            return carry

        lax.fori_loop(0, NITER, chunk, 0)
        plsc.subcore_barrier()
        pltpu.sync_copy(agg_sh.at[pl.ds(rbase, ROWS_PT)],
                        agg_out.at[pl.ds(l * N + rbase, ROWS_PT)])

    for i in range(5):
        l = i + 5 * c
        if i == 4:
            @pl.when(c == 0)
            def _():
                run_layer(l)
        else:
            run_layer(l)

    @pl.when(c == 1)
    def _():
        pltpu.sync_copy(zcnt.at[pl.ds(rbase, ROWS_PT)],
                        cnt_sh.at[pl.ds(rbase, ROWS_PT)])
        pltpu.sync_copy(ones_k, ones_v)
        plsc.subcore_barrier()

        def cchunk(it, carry):
            base = ebase + it * K
            pltpu.sync_copy(edges.at[1, pl.ds(base, K)], dst_v)
            pltpu.sync_copy(ones_v, cnt_sh.at[dst_v], add=True)
            return carry

        lax.fori_loop(0, NITER, cchunk, 0)
        plsc.subcore_barrier()
        pltpu.sync_copy(cnt_sh.at[pl.ds(rbase, ROWS_PT)],
                        cnt_out.at[pl.ds(rbase, ROWS_PT)])


_sc_segsum = pl.kernel(
    _sc_segsum_body,
    out_type=(jax.ShapeDtypeStruct((NENC * N, D), jnp.float32),
              jax.ShapeDtypeStruct((N, 16), jnp.float32)),
    mesh=_sc_mesh,
    scratch_types=[
        pltpu.VMEM_SHARED((N, D), jnp.float32),
        pltpu.VMEM_SHARED((N, 16), jnp.float32),
        pltpu.VMEM((K,), jnp.int32),
        pltpu.VMEM((K,), jnp.int32),
        pltpu.VMEM((K, D), jnp.float32),
        pltpu.VMEM((K, 16), jnp.float32),
        pltpu.SemaphoreType.DMA,
    ],
)


BN = 2000
NBLK = N // BN


def _enc_body(agg_ref, cnt_ref, x_ref, wlT_ref, bl_ref, wrT_ref,
              out_ref, acc_ref):
    b = pl.program_id(1)

    @pl.when(b == 0)
    def _():
        acc_ref[...] = jnp.zeros_like(acc_ref)

    inv = 1.0 / jnp.maximum(cnt_ref[...], 1.0)
    mean = agg_ref[0] * inv
    h = mean @ wlT_ref[...] + bl_ref[...] + x_ref[0] @ wrT_ref[...]
    h = jnp.maximum(h, 0.0)
    acc_ref[...] += jnp.sum(h, axis=0, keepdims=True)

    @pl.when(b == NBLK - 1)
    def _():
        out_ref[0] = acc_ref[...] * (1.0 / N)


_enc_call = pl.pallas_call(
    _enc_body,
    grid=(NENC, NBLK),
    in_specs=[
        pl.BlockSpec((1, BN, D), lambda l, b: (l, b, 0)),
        pl.BlockSpec((BN, 1), lambda l, b: (b, 0)),
        pl.BlockSpec((1, BN, D), lambda l, b: (l, b, 0)),
        pl.BlockSpec((D, H), lambda l, b: (0, 0)),
        pl.BlockSpec((1, H), lambda l, b: (0, 0)),
        pl.BlockSpec((D, H), lambda l, b: (0, 0)),
    ],
    out_specs=pl.BlockSpec((1, 1, H), lambda l, b: (l, 0, 0)),
    out_shape=jax.ShapeDtypeStruct((NENC, 1, H), jnp.float32),
    scratch_shapes=[pltpu.VMEM((1, H), jnp.float32)],
)


def _head_body(ns_ref, wihT_ref, whhT_ref, bih_ref, bhh_ref,
               muT_ref, mub_ref, lvT_ref, lvb_ref, eps_ref,
               dwzT_ref, dwcT_ref, db_ref,
               dlT_ref, dbl_ref, drT_ref,
               mu_ref, lv_ref, rows_ref):
    h = jnp.zeros((1, H), jnp.float32)
    for t in range(NL):
        gi = ns_ref[t:t + 1, :] @ wihT_ref[...] + bih_ref[...]
        gh = h @ whhT_ref[...] + bhh_ref[...]
        r = jax.nn.sigmoid(gi[:, :H] + gh[:, :H])
        z = jax.nn.sigmoid(gi[:, H:2 * H] + gh[:, H:2 * H])
        n = jnp.tanh(gi[:, 2 * H:] + r * gh[:, 2 * H:])
        h = (1.0 - z) * n + z * h
    mu = h @ muT_ref[...] + mub_ref[...]
    logvar = h @ lvT_ref[...] + lvb_ref[...]
    zlat = mu + eps_ref[...] * jnp.exp(0.5 * logvar)
    ctx = zlat @ dwzT_ref[...] + db_ref[...]
    for t in range(NENC):
        ctx = ctx + ns_ref[t:t + 1, :] @ dwcT_ref[t]
    dec_in = jnp.maximum(ctx, 0.0)
    base = dec_in @ drT_ref[...] + dbl_ref[...]
    add = dec_in @ dlT_ref[...]
    mu_ref[...] = mu
    lv_ref[...] = logvar
    rows_ref[0:1, :] = base + add
    rows_ref[1:2, :] = base


_head_call = pl.pallas_call(
    _head_body,
    out_shape=(jax.ShapeDtypeStruct((1, LAT), jnp.float32),
               jax.ShapeDtypeStruct((1, LAT), jnp.float32),
               jax.ShapeDtypeStruct((2, D), jnp.float32)),
)


BN2 = 500
NBLK2 = N // BN2


def _recon_body(rows_ref, cnt_ref, out_ref):
    sel = cnt_ref[...] > 0.0
    row = jnp.where(sel, rows_ref[0:1, :], rows_ref[1:2, :])
    out_ref[...] = jnp.broadcast_to(row[None], (NREC, BN2, D))


_recon_call = pl.pallas_call(
    _recon_body,
    grid=(NBLK2,),
    in_specs=[
        pl.BlockSpec((2, D), lambda b: (0, 0)),
        pl.BlockSpec((BN2, 1), lambda b: (b, 0)),
    ],
    out_specs=pl.BlockSpec((NREC, BN2, D), lambda b: (0, b, 0)),
    out_shape=jax.ShapeDtypeStruct((NREC, N, D), jnp.float32),
)


def kernel(x_layers, edge_index, mask, enc_Wl, enc_bl, enc_Wr,
           gru_Wih, gru_Whh, gru_bih, gru_bhh,
           mu_W, mu_b, lv_W, lv_b, dec_W, dec_b,
           dgnn_Wl, dgnn_bl, dgnn_Wr):
    f32 = jnp.float32
    x_flat = x_layers.reshape(NL * N, D)
    zrow = jnp.zeros((N, D), f32)
    zcnt = jnp.zeros((N, 16), f32)
    ones_k = jnp.ones((K, 16), f32)

    agg_flat, cnt16 = _sc_segsum(x_flat, edge_index, zrow, zcnt, ones_k)
    agg = agg_flat.reshape(NENC, N, D)
    cnt = cnt16[:, :1]

    sums = _enc_call(agg, cnt, x_layers[:NENC], enc_Wl.T,
                     enc_bl.reshape(1, H), enc_Wr.T)
    node_seq = jnp.concatenate(
        [sums.reshape(NENC, H), jnp.zeros((NL - NENC, H), f32)], axis=0)

    eps = jax.random.normal(jax.random.key(42), (LAT,), f32).reshape(1, LAT)
    dec_Wz = dec_W[:, :LAT].T
    dec_Wc = dec_W[:, LAT:LAT + NENC * H].T.reshape(NENC, H, H)
    mu2, lv2, rows2 = _head_call(
        node_seq, gru_Wih.T, gru_Whh.T,
        gru_bih.reshape(1, 3 * H), gru_bhh.reshape(1, 3 * H),
        mu_W.T, mu_b.reshape(1, LAT), lv_W.T, lv_b.reshape(1, LAT), eps,
        dec_Wz, dec_Wc, dec_b.reshape(1, H),
        dgnn_Wl.T, dgnn_bl.reshape(1, D), dgnn_Wr.T)

    recon = _recon_call(rows2, cnt)
    return recon, mu2.reshape(LAT), lv2.reshape(LAT)


# speedup vs baseline: 3.5591x; 3.5591x over previous
import functools

import jax
import jax.numpy as jnp
from jax import lax
from jax.experimental import pallas as pl
from jax.experimental.pallas import tpu as pltpu
from jax.experimental.pallas import tpu_sc as plsc

N = 10000
E = 320000
D = 128
H = 128
LAT = 64
G = 20
NL = 12
NENC = 9
NREC = 11

K = 80
TILES = 16
EPT = E // TILES
NITER = EPT // K
RCHUNK = 1000
RTILES = N // RCHUNK

def _sc_segsum_body(x_flat, src_e, dst_e, zrow, ones_r,
                    agg_out,
                    agg_sh, src_v, dst_v, rows_v, gsem):
    c = lax.axis_index("c")
    s = lax.axis_index("s")
    ebase = s * EPT
    rbase = s * RCHUNK

    def gather_chunks(l):
        def chunk(it, carry):
            base = ebase + it * K
            pltpu.sync_copy(src_e.at[pl.ds(base, K)], src_v)
            pltpu.sync_copy(dst_e.at[pl.ds(base, K)], dst_v)
            for j in range(K // 16):
                sl = pl.ds(j * 16, 16)
                src_v[sl] = src_v[sl] + l * N
            pltpu.async_copy(x_flat.at[src_v], rows_v, gsem).wait()
            pltpu.sync_copy(rows_v, agg_sh.at[dst_v], add=True)
            return carry

        lax.fori_loop(0, NITER, chunk, 0)

    def ones_chunks():
        def chunk(it, carry):
            base = ebase + it * K
            pltpu.sync_copy(dst_e.at[pl.ds(base, K)], dst_v)
            pltpu.sync_copy(rows_v, agg_sh.at[dst_v], add=True)
            return carry

        lax.fori_loop(0, NITER, chunk, 0)

    for i in range(5):
        l = i + 5 * c

        @pl.when(s < RTILES)
        def _():
            pltpu.sync_copy(zrow.at[pl.ds(rbase, RCHUNK)],
                            agg_sh.at[pl.ds(rbase, RCHUNK)])
        if i == 4:
            @pl.when(c == 1)
            def _():
                pltpu.sync_copy(ones_r, rows_v)
        plsc.subcore_barrier()

        if i == 4:
            @pl.when(c == 0)
            def _():
                gather_chunks(l)

            @pl.when(c == 1)
            def _():
                ones_chunks()
        else:
            gather_chunks(l)
        plsc.subcore_barrier()

        @pl.when(s < RTILES)
        def _():
            pltpu.sync_copy(agg_sh.at[pl.ds(rbase, RCHUNK)],
                            agg_out.at[pl.ds(l * N + rbase, RCHUNK)])


@functools.cache
def _get_sc_segsum():
    mesh = plsc.VectorSubcoreMesh(core_axis_name="c", subcore_axis_name="s",
                                  num_cores=2, num_subcores=TILES)
    return pl.kernel(
        _sc_segsum_body,
        out_type=jax.ShapeDtypeStruct(((NENC + 1) * N, D), jnp.float32),
        mesh=mesh,
        scratch_types=[
            pltpu.VMEM_SHARED((N, D), jnp.float32),
            pltpu.VMEM((K,), jnp.int32),
            pltpu.VMEM((K,), jnp.int32),
            pltpu.VMEM((K, D), jnp.float32),
            pltpu.SemaphoreType.DMA,
        ],
    )


BN = 2000
NBLK = N // BN


def _enc_body(agg_ref, cnt_ref, x_ref, wlT_ref, bl_ref, wrT_ref,
              out_ref, acc_ref):
    b = pl.program_id(1)

    @pl.when(b == 0)
    def _():
        acc_ref[...] = jnp.zeros_like(acc_ref)

    inv = 1.0 / jnp.maximum(cnt_ref[...], 1.0)
    mean = agg_ref[0] * inv
    h = mean @ wlT_ref[...] + bl_ref[...] + x_ref[0] @ wrT_ref[...]
    h = jnp.maximum(h, 0.0)
    acc_ref[...] += jnp.sum(h, axis=0, keepdims=True)

    @pl.when(b == NBLK - 1)
    def _():
        out_ref[0] = acc_ref[...] * (1.0 / N)


_enc_call = pl.pallas_call(
    _enc_body,
    grid=(NENC, NBLK),
    in_specs=[
        pl.BlockSpec((1, BN, D), lambda l, b: (l, b, 0)),
        pl.BlockSpec((BN, 1), lambda l, b: (b, 0)),
        pl.BlockSpec((1, BN, D), lambda l, b: (l, b, 0)),
        pl.BlockSpec((D, H), lambda l, b: (0, 0)),
        pl.BlockSpec((1, H), lambda l, b: (0, 0)),
        pl.BlockSpec((D, H), lambda l, b: (0, 0)),
    ],
    out_specs=pl.BlockSpec((1, 1, H), lambda l, b: (l, 0, 0)),
    out_shape=jax.ShapeDtypeStruct((NENC, 1, H), jnp.float32),
    scratch_shapes=[pltpu.VMEM((1, H), jnp.float32)],
)


def _head_body(ns_ref, wihT_ref, whhT_ref, bih_ref, bhh_ref,
               muT_ref, mub_ref, lvT_ref, lvb_ref, eps_ref,
               dwzT_ref, dwcT_ref, db_ref,
               dlT_ref, dbl_ref, drT_ref,
               mu_ref, lv_ref, rows_ref):
    h = jnp.zeros((1, H), jnp.float32)
    for t in range(NL):
        gi = ns_ref[t:t + 1, :] @ wihT_ref[...] + bih_ref[...]
        gh = h @ whhT_ref[...] + bhh_ref[...]
        r = jax.nn.sigmoid(gi[:, :H] + gh[:, :H])
        z = jax.nn.sigmoid(gi[:, H:2 * H] + gh[:, H:2 * H])
        n = jnp.tanh(gi[:, 2 * H:] + r * gh[:, 2 * H:])
        h = (1.0 - z) * n + z * h
    mu = h @ muT_ref[...] + mub_ref[...]
    logvar = h @ lvT_ref[...] + lvb_ref[...]
    zlat = mu + eps_ref[...] * jnp.exp(0.5 * logvar)
    ctx = zlat @ dwzT_ref[...] + db_ref[...]
    for t in range(NENC):
        ctx = ctx + ns_ref[t:t + 1, :] @ dwcT_ref[t]
    dec_in = jnp.maximum(ctx, 0.0)
    base = dec_in @ drT_ref[...] + dbl_ref[...]
    add = dec_in @ dlT_ref[...]
    mu_ref[...] = mu
    lv_ref[...] = logvar
    rows_ref[0:1, :] = base + add
    rows_ref[1:2, :] = base


_head_call = pl.pallas_call(
    _head_body,
    out_shape=(jax.ShapeDtypeStruct((1, LAT), jnp.float32),
               jax.ShapeDtypeStruct((1, LAT), jnp.float32),
               jax.ShapeDtypeStruct((2, D), jnp.float32)),
)


BN2 = 1000
NBLK2 = N // BN2


def _recon_body(rows_ref, cnt_ref, out_ref):
    sel = cnt_ref[...] > 0.0
    row = jnp.where(sel, rows_ref[0:1, :], rows_ref[1:2, :])
    out_ref[...] = jnp.broadcast_to(row[None], (NREC, BN2, D))


_recon_call = pl.pallas_call(
    _recon_body,
    grid=(NBLK2,),
    in_specs=[
        pl.BlockSpec((2, D), lambda b: (0, 0)),
        pl.BlockSpec((BN2, 1), lambda b: (b, 0)),
    ],
    out_specs=pl.BlockSpec((NREC, BN2, D), lambda b: (0, b, 0)),
    out_shape=jax.ShapeDtypeStruct((NREC, N, D), jnp.float32),
)


def kernel(x_layers, edge_index, mask, enc_Wl, enc_bl, enc_Wr,
           gru_Wih, gru_Whh, gru_bih, gru_bhh,
           mu_W, mu_b, lv_W, lv_b, dec_W, dec_b,
           dgnn_Wl, dgnn_bl, dgnn_Wr):
    f32 = jnp.float32
    x_flat = x_layers.reshape(NL * N, D)
    zrow = jnp.zeros((N, D), f32)
    ones_r = jnp.ones((K, D), f32)

    out_flat = _get_sc_segsum()(x_flat, edge_index[0], edge_index[1],
                                zrow, ones_r)
    agg = out_flat[:NENC * N].reshape(NENC, N, D)
    cnt = out_flat[NENC * N:, :1]

    sums = _enc_call(agg, cnt, x_layers[:NENC], enc_Wl.T,
                     enc_bl.reshape(1, H), enc_Wr.T)
    node_seq = jnp.concatenate(
        [sums.reshape(NENC, H), jnp.zeros((NL - NENC, H), f32)], axis=0)

    eps = jax.random.normal(jax.random.key(42), (LAT,), f32).reshape(1, LAT)
    dec_Wz = dec_W[:, :LAT].T
    dec_Wc = dec_W[:, LAT:LAT + NENC * H].T.reshape(NENC, H, H)
    mu2, lv2, rows2 = _head_call(
        node_seq, gru_Wih.T, gru_Whh.T,
        gru_bih.reshape(1, 3 * H), gru_bhh.reshape(1, 3 * H),
        mu_W.T, mu_b.reshape(1, LAT), lv_W.T, lv_b.reshape(1, LAT), eps,
        dec_Wz, dec_Wc, dec_b.reshape(1, H),
        dgnn_Wl.T, dgnn_bl.reshape(1, D), dgnn_Wr.T)

    recon = _recon_call(rows2, cnt)
    return recon, mu2.reshape(LAT), lv2.reshape(LAT)
